# baseline (device time: 17349 ns/iter reference)
import jax
import jax.numpy as jnp
from jax import lax
from jax.experimental import pallas as pl
from jax.experimental.pallas import tpu as pltpu


def kernel(x, dy, gamma):
    del gamma
    m, d = x.shape
    BLK = 512
    rows_local = m // 4
    nblk = rows_local // BLK

    q = (2 * lax.axis_index("x") + lax.axis_index("y")).astype(jnp.int32)
    q_arr = jnp.reshape(q, (1,))

    def body(q_ref, x_ref, dy_ref, out_ref, acc_ref, comm_ref,
             send_sems, recv_sems):
        i = pl.program_id(0)
        my_x = lax.axis_index("x")
        my_y = lax.axis_index("y")
        my_z = lax.axis_index("z")
        partners = [
            (1 - my_x, my_y, my_z),
            (my_x, 1 - my_y, my_z),
            (my_x, my_y, 1 - my_z),
        ]
        barrier = pltpu.get_barrier_semaphore()

        @pl.when(i == 0)
        def _init():
            acc_ref[...] = jnp.zeros_like(acc_ref)
            for p in partners:
                pl.semaphore_signal(
                    barrier, inc=1,
                    device_id=p, device_id_type=pl.DeviceIdType.MESH,
                )

        xb16 = x_ref[...].astype(jnp.bfloat16)
        dy16 = dy_ref[...].astype(jnp.bfloat16)
        ones_d = jnp.ones((d, 1), jnp.bfloat16)
        dims = (((1,), (0,)), ((), ()))
        s1 = lax.dot_general(
            xb16, ones_d, dims, preferred_element_type=jnp.float32
        )
        s2 = lax.dot_general(
            xb16 * xb16, ones_d, dims, preferred_element_type=jnp.float32
        )
        mu = s1 / d
        var = s2 / d - mu * mu
        rstd = lax.rsqrt(var + 1e-5)
        xhat = (xb16 - mu.astype(jnp.bfloat16)) * rstd.astype(jnp.bfloat16)
        ones_r = jnp.ones((1, BLK), jnp.bfloat16)
        dgamma = lax.dot_general(
            ones_r, dy16 * xhat, dims, preferred_element_type=jnp.float32
        )
        dbeta = lax.dot_general(
            ones_r, dy16, dims, preferred_element_type=jnp.float32
        )
        acc_ref[...] += jnp.concatenate([dgamma, dbeta], axis=0)

        @pl.when(i == nblk - 1)
        def _allreduce():
            pl.semaphore_wait(barrier, 3)
            for s, p in enumerate(partners):
                rdma = pltpu.make_async_remote_copy(
                    src_ref=acc_ref,
                    dst_ref=comm_ref.at[s],
                    send_sem=send_sems.at[s],
                    recv_sem=recv_sems.at[s],
                    device_id=p,
                    device_id_type=pl.DeviceIdType.MESH,
                )
                rdma.start()
                rdma.wait()
                acc_ref[...] += comm_ref[s]

            out_ref[...] = acc_ref[...]

    grid_spec = pltpu.PrefetchScalarGridSpec(
        num_scalar_prefetch=1,
        grid=(nblk,),
        in_specs=[
            pl.BlockSpec((BLK, d), lambda i, q: (q[0] * nblk + i, 0)),
            pl.BlockSpec((BLK, d), lambda i, q: (q[0] * nblk + i, 0)),
        ],
        out_specs=pl.BlockSpec((2, d), lambda i, q: (0, 0)),
        scratch_shapes=[
            pltpu.VMEM((2, d), jnp.float32),
            pltpu.VMEM((3, 2, d), jnp.float32),
            pltpu.SemaphoreType.DMA((3,)),
            pltpu.SemaphoreType.DMA((3,)),
        ],
    )

    return pl.pallas_call(
        body,
        grid_spec=grid_spec,
        out_shape=jax.ShapeDtypeStruct((2, d), jnp.float32),
        compiler_params=pltpu.CompilerParams(
            dimension_semantics=("arbitrary",),
            collective_id=0,
        ),
    )(q_arr, x, dy)


# device time: 14701 ns/iter; 1.1801x vs baseline; 1.1801x over previous
import jax
import jax.numpy as jnp
from jax import lax
from jax.experimental import pallas as pl
from jax.experimental.pallas import tpu as pltpu

_OFFSETS = [
    (0, 0, 1), (0, 1, 0), (0, 1, 1),
    (1, 0, 0), (1, 0, 1), (1, 1, 0), (1, 1, 1),
]


def kernel(x, dy, gamma):
    del gamma
    m, d = x.shape
    BLK = 512
    rows_local = m // 4
    nblk = rows_local // BLK

    q = (2 * lax.axis_index("x") + lax.axis_index("y")).astype(jnp.int32)
    q_arr = jnp.reshape(q, (1,))

    def body(q_ref, x_ref, dy_ref, out_ref, acc_ref, comm_ref,
             send_sems, recv_sems):
        i = pl.program_id(0)
        my_x = lax.axis_index("x")
        my_y = lax.axis_index("y")
        my_z = lax.axis_index("z")
        peers = [
            (
                1 - my_x if ox else my_x,
                1 - my_y if oy else my_y,
                1 - my_z if oz else my_z,
            )
            for ox, oy, oz in _OFFSETS
        ]
        barrier = pltpu.get_barrier_semaphore()

        @pl.when(i == 0)
        def _init():
            acc_ref[...] = jnp.zeros_like(acc_ref)
            for p in peers:
                pl.semaphore_signal(
                    barrier, inc=1,
                    device_id=p, device_id_type=pl.DeviceIdType.MESH,
                )

        xb = x_ref[...]
        dyb = dy_ref[...]
        mu = jnp.mean(xb, axis=1, keepdims=True)
        xc = xb - mu
        var = jnp.mean(xc * xc, axis=1, keepdims=True)
        xhat = xc * lax.rsqrt(var + 1e-5)
        dgamma = jnp.sum(dyb * xhat, axis=0)
        dbeta = jnp.sum(dyb, axis=0)
        acc_ref[...] += jnp.concatenate(
            [dgamma[None, :], dbeta[None, :]], axis=0
        )

        @pl.when(i == nblk - 1)
        def _allreduce():
            pl.semaphore_wait(barrier, 7)
            rdmas = []
            for s, p in enumerate(peers):
                rdma = pltpu.make_async_remote_copy(
                    src_ref=acc_ref,
                    dst_ref=comm_ref.at[s],
                    send_sem=send_sems.at[s],
                    recv_sem=recv_sems.at[s],
                    device_id=p,
                    device_id_type=pl.DeviceIdType.MESH,
                )
                rdma.start()
                rdmas.append(rdma)
            for rdma in rdmas:
                rdma.wait_recv()
            total = acc_ref[...]
            for s in range(len(peers)):
                total += comm_ref[s]
            out_ref[...] = total
            for rdma in rdmas:
                rdma.wait_send()

    grid_spec = pltpu.PrefetchScalarGridSpec(
        num_scalar_prefetch=1,
        grid=(nblk,),
        in_specs=[
            pl.BlockSpec((BLK, d), lambda i, q: (q[0] * nblk + i, 0)),
            pl.BlockSpec((BLK, d), lambda i, q: (q[0] * nblk + i, 0)),
        ],
        out_specs=pl.BlockSpec((2, d), lambda i, q: (0, 0)),
        scratch_shapes=[
            pltpu.VMEM((2, d), jnp.float32),
            pltpu.VMEM((7, 2, d), jnp.float32),
            pltpu.SemaphoreType.DMA((7,)),
            pltpu.SemaphoreType.DMA((7,)),
        ],
    )

    return pl.pallas_call(
        body,
        grid_spec=grid_spec,
        out_shape=jax.ShapeDtypeStruct((2, d), jnp.float32),
        compiler_params=pltpu.CompilerParams(
            dimension_semantics=("arbitrary",),
            collective_id=0,
        ),
    )(q_arr, x, dy)
